# baseline (speedup 1.0000x reference)
import functools

import jax
import jax.numpy as jnp
from jax import lax
from jax.experimental import pallas as pl
from jax.experimental.pallas import tpu as pltpu
from jax.experimental.pallas import tpu_sc as plsc

_B = 16
_TIME = 2048
_NWIN = 1948
_WPT = 976
_CH_LEN = 1088
_S10_GROUPS = 67
_WIN_GROUPS = 61

_BINS = (
    ((0,), 5.0),
    ((10, 20), 10.0),
    ((30, 40, 50), 15.0),
    ((60, 70, 80, 90), 20.0),
)


def _body(va_hbm, out_hbm, va_v, ch0, ch1, s0, s1, outv):
    b = lax.axis_index("s")
    lane = lax.broadcasted_iota(jnp.int32, (16,), 0)
    one = jnp.full((16,), 1, jnp.int32)
    zero = jnp.full((16,), 0, jnp.int32)

    pltpu.sync_copy(va_hbm.at[pl.ds(b * (_TIME * 2), _TIME * 2)], va_v)

    for h in (0, 1):
        w0 = h * _WPT

        for c, ch in ((0, ch0), (1, ch1)):

            @plsc.parallel_loop(0, _CH_LEN // 16, unroll=2)
            def _(g, ch=ch, c=c, w0=w0):
                rows = jnp.minimum(w0 + g * 16 + 1 + lane, _TIME - 1)
                ch[pl.ds(g * 16, 16)] = plsc.load_gather(va_v, [rows * 2 + c])

        for ch, s in ((ch0, s0), (ch1, s1)):

            @plsc.parallel_loop(0, _S10_GROUPS, unroll=2)
            def _(g, ch=ch, s=s):
                base = g * 16
                acc = ch[pl.ds(base, 16)]
                for t in range(1, 10):
                    acc = acc + ch[pl.ds(base + t, 16)]
                s[pl.ds(base, 16)] = acc

        @plsc.parallel_loop(0, _WIN_GROUPS, unroll=2)
        def _(g):
            base = g * 16
            code = zero
            for c, s in ((0, s0), (1, s1)):
                for k, (offs, thr) in enumerate(_BINS):
                    acc = s[pl.ds(base + offs[0], 16)]
                    for o in offs[1:]:
                        acc = acc + s[pl.ds(base + o, 16)]
                    bit = jnp.where(acc >= thr, one, zero)
                    code = code + bit * (1 << (c * 4 + k))
            outv[pl.ds(base, 16)] = code

        pltpu.sync_copy(outv, out_hbm.at[pl.ds(b * _TIME + w0, _WPT)])


def kernel(va, emb_weight):
    del emb_weight
    mesh = plsc.VectorSubcoreMesh(
        core_axis_name="c", subcore_axis_name="s", num_cores=1, num_subcores=16
    )
    padded = pl.kernel(
        _body,
        out_type=jax.ShapeDtypeStruct((_B * _TIME,), jnp.int32),
        mesh=mesh,
        compiler_params=pltpu.CompilerParams(needs_layout_passes=False),
        scratch_types=[
            pltpu.VMEM((_TIME * 2,), jnp.float32),
            pltpu.VMEM((_CH_LEN,), jnp.float32),
            pltpu.VMEM((_CH_LEN,), jnp.float32),
            pltpu.VMEM((_CH_LEN,), jnp.float32),
            pltpu.VMEM((_CH_LEN,), jnp.float32),
            pltpu.VMEM((_WPT,), jnp.int32),
        ],
    )(va.reshape(-1))
    return padded.reshape(_B, _TIME)[:, :_NWIN]


# speedup vs baseline: 1.0154x; 1.0154x over previous
import functools

import jax
import jax.numpy as jnp
from jax import lax
from jax.experimental import pallas as pl
from jax.experimental.pallas import tpu as pltpu
from jax.experimental.pallas import tpu_sc as plsc

_B = 16
_TIME = 2048
_NWIN = 1948
_WPT = 976
_CH_LEN = 1088
_S10_GROUPS = 67
_WIN_GROUPS = 61

_BINS = (
    ((0,), 5.0),
    ((10, 20), 10.0),
    ((30, 40, 50), 15.0),
    ((60, 70, 80, 90), 20.0),
)


def _body(va_hbm, out_hbm, va_v, ch0, ch1, s0, s1, outv):
    b = lax.axis_index("s")
    h = lax.axis_index("c")
    w0 = h * _WPT
    lane = lax.broadcasted_iota(jnp.int32, (16,), 0)

    pltpu.sync_copy(va_hbm.at[pl.ds(b * (_TIME * 2), _TIME * 2)], va_v)

    for c, ch in ((0, ch0), (1, ch1)):

        @plsc.parallel_loop(0, _CH_LEN // 16, unroll=4)
        def _(g, ch=ch, c=c):
            rows = jnp.minimum(w0 + g * 16 + 1 + lane, _TIME - 1)
            ch[pl.ds(g * 16, 16)] = plsc.load_gather(va_v, [rows * 2 + c])

    for ch, s in ((ch0, s0), (ch1, s1)):

        @plsc.parallel_loop(0, _S10_GROUPS, unroll=4)
        def _(g, ch=ch, s=s):
            base = g * 16
            acc = ch[pl.ds(base, 16)]
            for t in range(1, 10):
                acc = acc + ch[pl.ds(base + t, 16)]
            s[pl.ds(base, 16)] = acc

    one = jnp.full((16,), 1, jnp.int32)
    zero = jnp.full((16,), 0, jnp.int32)

    @plsc.parallel_loop(0, _WIN_GROUPS, unroll=4)
    def _(g):
        base = g * 16
        code = zero
        for c, s in ((0, s0), (1, s1)):
            for k, (offs, thr) in enumerate(_BINS):
                acc = s[pl.ds(base + offs[0], 16)]
                for o in offs[1:]:
                    acc = acc + s[pl.ds(base + o, 16)]
                bit = jnp.where(acc >= thr, one, zero)
                code = code + bit * (1 << (c * 4 + k))
        outv[pl.ds(base, 16)] = code

    pltpu.sync_copy(outv, out_hbm.at[pl.ds(b * _TIME + w0, _WPT)])


def kernel(va, emb_weight):
    del emb_weight
    mesh = plsc.VectorSubcoreMesh(
        core_axis_name="c", subcore_axis_name="s", num_cores=2, num_subcores=16
    )
    padded = pl.kernel(
        _body,
        out_type=jax.ShapeDtypeStruct((_B * _TIME,), jnp.int32),
        mesh=mesh,
        compiler_params=pltpu.CompilerParams(needs_layout_passes=False),
        scratch_types=[
            pltpu.VMEM((_TIME * 2,), jnp.float32),
            pltpu.VMEM((_CH_LEN,), jnp.float32),
            pltpu.VMEM((_CH_LEN,), jnp.float32),
            pltpu.VMEM((_CH_LEN,), jnp.float32),
            pltpu.VMEM((_CH_LEN,), jnp.float32),
            pltpu.VMEM((_WPT,), jnp.int32),
        ],
    )(va.reshape(-1))
    return padded.reshape(_B, _TIME)[:, :_NWIN]


# speedup vs baseline: 1.0197x; 1.0043x over previous
import functools

import jax
import jax.numpy as jnp
from jax import lax
from jax.experimental import pallas as pl
from jax.experimental.pallas import tpu as pltpu
from jax.experimental.pallas import tpu_sc as plsc

_B = 16
_TIME = 2048
_NWIN = 1948
_WPT = 976
_CH_LEN = 1088
_S10_GROUPS = 67
_WIN_GROUPS = 61

_BINS = (
    ((0,), 5.0),
    ((10, 20), 10.0),
    ((30, 40, 50), 15.0),
    ((60, 70, 80, 90), 20.0),
)


def _body(va_hbm, out_hbm, va_v, ch0, ch1, s0, s1, outv):
    b = lax.axis_index("s")
    h = lax.axis_index("c")
    w0 = h * _WPT
    lane = lax.broadcasted_iota(jnp.int32, (16,), 0)

    pltpu.sync_copy(va_hbm.at[pl.ds(b * (_TIME * 2), _TIME * 2)], va_v)

    for c, ch in ((0, ch0), (1, ch1)):

        @plsc.parallel_loop(0, _CH_LEN // 16, unroll=2)
        def _(g, ch=ch, c=c):
            rows = jnp.minimum(w0 + g * 16 + 1 + lane, _TIME - 1)
            ch[pl.ds(g * 16, 16)] = plsc.load_gather(va_v, [rows * 2 + c])

    for ch, s in ((ch0, s0), (ch1, s1)):

        @plsc.parallel_loop(0, _S10_GROUPS, unroll=2)
        def _(g, ch=ch, s=s):
            base = g * 16
            acc = ch[pl.ds(base, 16)]
            for t in range(1, 10):
                acc = acc + ch[pl.ds(base + t, 16)]
            s[pl.ds(base, 16)] = acc

    one = jnp.full((16,), 1, jnp.int32)
    zero = jnp.full((16,), 0, jnp.int32)

    @plsc.parallel_loop(0, _WIN_GROUPS, unroll=2)
    def _(g):
        base = g * 16
        code = zero
        for c, s in ((0, s0), (1, s1)):
            for k, (offs, thr) in enumerate(_BINS):
                acc = s[pl.ds(base + offs[0], 16)]
                for o in offs[1:]:
                    acc = acc + s[pl.ds(base + o, 16)]
                bit = jnp.where(acc >= thr, one, zero)
                code = code + bit * (1 << (c * 4 + k))
        outv[pl.ds(base, 16)] = code

    pltpu.sync_copy(outv, out_hbm.at[pl.ds(b * _TIME + w0, _WPT)])


def kernel(va, emb_weight):
    del emb_weight
    mesh = plsc.VectorSubcoreMesh(
        core_axis_name="c", subcore_axis_name="s", num_cores=2, num_subcores=16
    )
    padded = pl.kernel(
        _body,
        out_type=jax.ShapeDtypeStruct((_B * _TIME,), jnp.int32),
        mesh=mesh,
        compiler_params=pltpu.CompilerParams(
            needs_layout_passes=False,
            disable_bounds_checks=True,
            disable_semaphore_checks=True,
        ),
        scratch_types=[
            pltpu.VMEM((_TIME * 2,), jnp.float32),
            pltpu.VMEM((_CH_LEN,), jnp.float32),
            pltpu.VMEM((_CH_LEN,), jnp.float32),
            pltpu.VMEM((_CH_LEN,), jnp.float32),
            pltpu.VMEM((_CH_LEN,), jnp.float32),
            pltpu.VMEM((_WPT,), jnp.int32),
        ],
    )(va.reshape(-1))
    return padded.reshape(_B, _TIME)[:, :_NWIN]
